# baseline (speedup 1.0000x reference)
import functools

import jax
import jax.numpy as jnp
from jax import lax
from jax.experimental import pallas as pl
from jax.experimental.pallas import tpu as pltpu
from jax.experimental.pallas import tpu_sc as plsc

NC = 2
NPR = 3

_LOG1P = (-0.002260995288484514, 0.015055349381361889, -0.04705113446017369,
          0.09475556299860108, -0.1453396417991718, 0.19351750061793127,
          -0.24872052840014142, 0.33318192091083904, -0.49999062475207734,
          0.9999997699016362, 9.473307568734822e-10)


def _log1p_poly(u):
    acc = jnp.full(u.shape, _LOG1P[0], jnp.float32)
    for c in _LOG1P[1:]:
        acc = acc * u + jnp.float32(c)
    return acc


def _sc_scan_hist(h_ref, nblk, k):
    i16 = lax.broadcasted_iota(jnp.int32, (16,), 0)

    def step(jj, carry):
        before, found, bsel, krem = carry
        j = nblk - 1 - jj
        v = h_ref[pl.ds(j * 16, 16)]
        rv = lax.rev(v, (0,))
        cums = plsc.cumsum(rv)
        tot = jnp.sum(v)
        after = before + tot
        crossed = jnp.logical_and(jnp.logical_and(before < k, after >= k),
                                  found == 0)
        cc = before + cums
        lane = jnp.min(jnp.where(cc >= k, i16, 16))
        sat = jnp.sum(jnp.where(i16 == lane, cc, 0.0))
        hv = jnp.sum(jnp.where(i16 == lane, rv, 0.0))
        nb = j * 16 + (15 - lane)
        bsel = jnp.where(crossed, nb, bsel)
        krem = jnp.where(crossed, k - (sat - hv), krem)
        found = jnp.where(crossed, 1, found)
        return (after, found, bsel, krem)

    endcum, found, bsel, krem = lax.fori_loop(
        0, nblk, step, (jnp.float32(0.0), jnp.int32(0), jnp.int32(0),
                        jnp.float32(0.0)))
    return found, bsel, krem, endcum


def _sc_settle(hs_ref, off, nwords, hv_ref, expected):
    def total_of(_):
        def ssum(i, a):
            return a + jnp.sum(hv_ref[pl.ds(i * 16, 16)])

        return lax.fori_loop(0, nwords // 16, ssum, jnp.float32(0.0))

    def body(_):
        pltpu.sync_copy(hs_ref.at[pl.ds(off, nwords)],
                        hv_ref.at[pl.ds(0, nwords)])
        return total_of(0)

    lax.while_loop(lambda t: t < expected, body, jnp.float32(-1.0))


def _sc_body(cp_hbm, t_hbm, out_hbm, cp_v, t_v, loss_v, ce_v, bin_v, one_v,
             hv, stg, h1s, h2s, h3s):
    c = lax.axis_index("c")
    s = lax.axis_index("s")

    @pl.when(jnp.logical_and(c == 0, s < 8))
    def _run():
        row = s
        pltpu.sync_copy(cp_hbm.at[row], cp_v)
        pltpu.sync_copy(t_hbm.at[row], t_v)

        zf = jnp.zeros((16,), jnp.float32)
        of = jnp.ones((16,), jnp.float32)

        def zloop(i, _):
            one_v[pl.ds(i * 16, 16)] = of

            @pl.when(i < 128)
            def _z1():
                hv[pl.ds(i * 16, 16)] = zf

            return 0

        lax.fori_loop(0, 1024, zloop, 0)
        pltpu.sync_copy(hv, h1s.at[pl.ds(row * 2048, 2048)])
        pltpu.sync_copy(hv.at[pl.ds(0, 1056)], h2s.at[pl.ds(row * 1056, 1056)])
        pltpu.sync_copy(hv.at[pl.ds(0, 1056)], h3s.at[pl.ds(row * 1056, 1056)])

        def p1(q, carry):
            npos, spos = carry
            ti = q >> 3
            lo = (q & 7) * 16
            c0 = cp_v[2 * ti, pl.ds(lo, 16)]
            c1 = cp_v[2 * ti + 1, pl.ds(lo, 16)]
            tv = t_v[pl.ds(q * 16, 16)]
            pos = jnp.clip(tv, 0, 1) > 0
            dmax = jnp.maximum(c0, c1)
            gat = jnp.where(pos, c1, c0)
            ce = dmax - gat + _log1p_poly(jnp.exp(-jnp.abs(c0 - c1)))
            loss = jnp.where(pos, 0.0, ce)
            ce_v[pl.ds(q * 16, 16)] = ce
            loss_v[pl.ds(q * 16, 16)] = loss
            bits = lax.bitcast_convert_type(loss, jnp.int32)
            bin_v[pl.ds(q * 16, 16)] = (
                lax.shift_right_logical(bits, 20) + row * 2048)
            return (npos + jnp.sum(pos.astype(jnp.float32)),
                    spos + jnp.sum(jnp.where(pos, ce, 0.0)))

        num_pos, sum_pos = lax.fori_loop(0, 1024, p1,
                                         (jnp.float32(0.0), jnp.float32(0.0)))
        pltpu.sync_copy(one_v, h1s.at[bin_v], add=True)
        _sc_settle(h1s, row * 2048, 2048, hv, jnp.float32(16384.0))
        num_neg = jnp.minimum(jnp.float32(NPR) * num_pos, jnp.float32(16383.0))

        f1, b1, k1, _e1 = _sc_scan_hist(hv, 128, num_neg)

        def p2(q, _):
            bits = lax.bitcast_convert_type(loss_v[pl.ds(q * 16, 16)],
                                            jnp.int32)
            m = lax.shift_right_logical(bits, 20) == b1
            b2v = jnp.bitwise_and(lax.shift_right_logical(bits, 10), 1023)
            bin_v[pl.ds(q * 16, 16)] = jnp.where(m, b2v, 1024) + row * 1056
            return 0

        lax.fori_loop(0, 1024, p2, 0)
        pltpu.sync_copy(one_v, h2s.at[bin_v], add=True)
        _sc_settle(h2s, row * 1056, 1056, hv, jnp.float32(16384.0))
        f2, b2, k2, _e2 = _sc_scan_hist(hv, 64, k1)

        hi21 = jnp.bitwise_or(b1 << 10, b2)

        def p3(q, _):
            bits = lax.bitcast_convert_type(loss_v[pl.ds(q * 16, 16)],
                                            jnp.int32)
            m = lax.shift_right_logical(bits, 10) == hi21
            bin_v[pl.ds(q * 16, 16)] = jnp.where(
                m, jnp.bitwise_and(bits, 1023), 1024) + row * 1056
            return 0

        lax.fori_loop(0, 1024, p3, 0)
        pltpu.sync_copy(one_v, h3s.at[bin_v], add=True)
        _sc_settle(h3s, row * 1056, 1056, hv, jnp.float32(16384.0))
        f3, b3, k3, _e3 = _sc_scan_hist(hv, 64, k2)

        vbits = jnp.where(f3, (b1 << 20) | (b2 << 10) | b3,
                          jnp.int32(0x7F800000))
        vvec = lax.bitcast_convert_type(jnp.broadcast_to(vbits, (16,)),
                                        jnp.float32)

        def p4(q, acc):
            loss = loss_v[pl.ds(q * 16, 16)]
            ce = ce_v[pl.ds(q * 16, 16)]
            return acc + jnp.sum(jnp.where(loss > vvec, ce, 0.0))

        sum_gt = lax.fori_loop(0, 1024, p4, jnp.float32(0.0))

        vstar = jnp.max(vvec)
        tie = jnp.where(jnp.logical_and(f3 > 0, k3 > 0),
                        vstar * k3, 0.0)
        cls_row = sum_pos + sum_gt + tie

        i16 = lax.broadcasted_iota(jnp.int32, (16,), 0)
        stg[pl.ds(0, 16)] = jnp.where(i16 == 0, cls_row,
                                      jnp.where(i16 == 1, num_pos, 0.0))

        def zs(i, _):
            stg[pl.ds((i + 1) * 16, 16)] = jnp.zeros((16,), jnp.float32)
            return 0

        lax.fori_loop(0, 7, zs, 0)
        pltpu.sync_copy(stg, out_hbm.at[row])


def _sc_cls(cp, tg):
    mesh = plsc.VectorSubcoreMesh(core_axis_name="c", subcore_axis_name="s")
    kern = functools.partial(
        pl.kernel,
        mesh=mesh,
        compiler_params=pltpu.CompilerParams(needs_layout_passes=False),
        out_type=jax.ShapeDtypeStruct((8, 128), jnp.float32),
        scratch_types=[
            pltpu.VMEM((256, 128), jnp.float32),
            pltpu.VMEM((16384,), jnp.int32),
            pltpu.VMEM((16384,), jnp.float32),
            pltpu.VMEM((16384,), jnp.float32),
            pltpu.VMEM((16384,), jnp.int32),
            pltpu.VMEM((16384,), jnp.float32),
            pltpu.VMEM((2048,), jnp.float32),
            pltpu.VMEM((128,), jnp.float32),
            pltpu.VMEM_SHARED((16384,), jnp.float32),
            pltpu.VMEM_SHARED((8448,), jnp.float32),
            pltpu.VMEM_SHARED((8448,), jnp.float32),
        ],
    )(_sc_body)
    return kern(cp, tg)


def _tc_body(x_ref, m_ref, lp_ref, lt_ref, t8_ref, out_ref, acc_ref, *,
             grid_i, grid_j, gts_den):
    i = pl.program_id(0)
    j = pl.program_id(1)
    step = i * grid_j + j
    last = grid_i * grid_j - 1

    @pl.when(step == 0)
    def _init():
        acc_ref[0] = 0.0
        acc_ref[1] = 0.0

    x = x_ref[...].reshape(2048, 128)
    bce_sp = jnp.sum(jnp.maximum(x, 0.0) + jnp.log1p(jnp.exp(-jnp.abs(x))))
    xp = x.reshape(1024, 256)
    mf = m_ref[...].reshape(1024, 128) > 0
    gathered = jnp.sum(jnp.where(mf, xp[:, 128:], xp[:, :128]))
    acc_ref[0] = acc_ref[0] + (bce_sp - gathered)

    d = lp_ref[...] - lt_ref[...]
    ad = jnp.abs(d)
    sl1 = jnp.where(ad < 1.0, 0.5 * d * d, ad - 0.5)
    posl = (jnp.clip(t8_ref[...], 0, 1) > 0)[:, None, :]
    acc_ref[1] = acc_ref[1] + jnp.sum(jnp.where(posl, sl1, 0.0))

    @pl.when(step == last)
    def _fin():
        lane = lax.broadcasted_iota(jnp.int32, (1, 128), 1)
        vec = jnp.where(lane == 0, acc_ref[1],
                        jnp.where(lane == 1, acc_ref[0] / gts_den, 0.0))
        out_ref[...] = vec


def _tc_dense(xg, gts_masks, lp, lt, cls_targets, B, K, A, H, W, L):
    GI, GJ = 4, 4
    BB = B // GI
    awb = A // (GI * GJ)
    rows = H * (W // 128) * NC
    in_specs = [
        pl.BlockSpec((1, BB, rows, 128), lambda i, j: (j, i, 0, 0)),
        pl.BlockSpec((BB, H, W), lambda i, j: (i, 0, 0)),
        pl.BlockSpec((B, K, awb), lambda i, j: (0, 0, i * GJ + j)),
        pl.BlockSpec((B, K, awb), lambda i, j: (0, 0, i * GJ + j)),
        pl.BlockSpec((B, awb), lambda i, j: (0, i * GJ + j)),
    ]
    body = functools.partial(_tc_body, grid_i=GI, grid_j=GJ,
                             gts_den=float(L * B * H * W * NC))
    return pl.pallas_call(
        body,
        grid=(GI, GJ),
        in_specs=in_specs,
        out_specs=pl.BlockSpec((1, 128), lambda i, j: (0, 0)),
        out_shape=jax.ShapeDtypeStruct((1, 128), jnp.float32),
        scratch_shapes=[pltpu.SMEM((2,), jnp.float32)],
        compiler_params=pltpu.CompilerParams(
            dimension_semantics=("arbitrary", "arbitrary")),
    )(xg, gts_masks, lp, lt, cls_targets)


def _combine_body(sc_ref, tc_ref, out_ref):
    scv = sc_ref[...]
    l16 = lax.broadcasted_iota(jnp.int32, (8, 128), 1)
    cls_sum = jnp.sum(jnp.where(l16 == 0, scv, 0.0))
    n_tot = jnp.sum(jnp.where(l16 == 1, scv, 0.0))
    tcv = tc_ref[...]
    lane = lax.broadcasted_iota(jnp.int32, (1, 128), 1)
    loc_sum = jnp.sum(jnp.where(lane == 0, tcv, 0.0))
    gts_loss = jnp.sum(jnp.where(lane == 1, tcv, 0.0))
    vec = jnp.where(lane == 0, loc_sum / n_tot,
                    jnp.where(lane == 1, cls_sum / n_tot,
                              jnp.where(lane == 2, gts_loss, 0.0)))
    out_ref[...] = vec


def kernel(loc_preds, loc_targets, cls_preds, cls_targets, global_text_segs,
           gts_masks):
    B, A, K = loc_preds.shape
    L = global_text_segs.shape[0]
    H, W = gts_masks.shape[1:]

    lp = jnp.transpose(loc_preds, (0, 2, 1))
    lt = jnp.transpose(loc_targets, (0, 2, 1))
    cp = cls_preds.reshape(B, A // 128, 128, NC).transpose(0, 1, 3, 2)
    cp = cp.reshape(B, (A // 128) * NC, 128)
    xg = global_text_segs.reshape(L, B, H, W // 128, 128, NC)
    xg = xg.transpose(0, 1, 2, 3, 5, 4).reshape(L, B, H * (W // 128) * NC, 128)

    sc_out = _sc_cls(cp, cls_targets)
    tc_out = _tc_dense(xg, gts_masks, lp, lt, cls_targets, B, K, A, H, W, L)

    out = pl.pallas_call(
        _combine_body,
        out_shape=jax.ShapeDtypeStruct((1, 128), jnp.float32),
    )(sc_out, tc_out)
    return (out[0, 0], out[0, 1], out[0, 2])


# speedup vs baseline: 1.4119x; 1.4119x over previous
import functools

import jax
import jax.numpy as jnp
from jax import lax
from jax.experimental import pallas as pl
from jax.experimental.pallas import tpu as pltpu
from jax.experimental.pallas import tpu_sc as plsc

NC = 2
NPR = 3

_LOG1P = (-0.002260995288484514, 0.015055349381361889, -0.04705113446017369,
          0.09475556299860108, -0.1453396417991718, 0.19351750061793127,
          -0.24872052840014142, 0.33318192091083904, -0.49999062475207734,
          0.9999997699016362, 9.473307568734822e-10)


def _log1p_poly(u):
    acc = jnp.full(u.shape, _LOG1P[0], jnp.float32)
    for c in _LOG1P[1:]:
        acc = acc * u + jnp.float32(c)
    return acc


def _sc_scan_hist(h_ref, nblk, k):
    i16 = lax.broadcasted_iota(jnp.int32, (16,), 0)

    def step(jj, carry):
        before, found, bsel, krem = carry
        j = nblk - 1 - jj
        v = h_ref[pl.ds(j * 16, 16)]
        rv = lax.rev(v, (0,))
        cums = plsc.cumsum(rv)
        tot = jnp.sum(v)
        after = before + tot
        crossed = jnp.logical_and(jnp.logical_and(before < k, after >= k),
                                  found == 0)
        cc = before + cums
        lane = jnp.min(jnp.where(cc >= k, i16, 16))
        sat = jnp.sum(jnp.where(i16 == lane, cc, 0.0))
        hv = jnp.sum(jnp.where(i16 == lane, rv, 0.0))
        nb = j * 16 + (15 - lane)
        bsel = jnp.where(crossed, nb, bsel)
        krem = jnp.where(crossed, k - (sat - hv), krem)
        found = jnp.where(crossed, 1, found)
        return (after, found, bsel, krem)

    endcum, found, bsel, krem = lax.fori_loop(
        0, nblk, step, (jnp.float32(0.0), jnp.int32(0), jnp.int32(0),
                        jnp.float32(0.0)))
    return found, bsel, krem, endcum


def _sc_body(cp_hbm, t_hbm, out_hbm, cp_v, t_v, loss_v, ce_v, h1, h2, h3,
             stg):
    c = lax.axis_index("c")
    s = lax.axis_index("s")

    @pl.when(jnp.logical_and(c == 0, s < 8))
    def _run():
        row = s
        pltpu.sync_copy(cp_hbm.at[row], cp_v)
        pltpu.sync_copy(t_hbm.at[row], t_v)

        zf = jnp.zeros((16,), jnp.float32)

        def zloop(i, _):
            h1[pl.ds(i * 16, 16)] = zf

            @pl.when(i < 66)
            def _z2():
                h2[pl.ds(i * 16, 16)] = zf
                h3[pl.ds(i * 16, 16)] = zf

            return 0

        lax.fori_loop(0, 128, zloop, 0)
        ones = jnp.ones((16,), jnp.float32)

        def p1(q, carry):
            npos, spos = carry
            ti = q >> 3
            lo = (q & 7) * 16
            c0 = cp_v[2 * ti, pl.ds(lo, 16)]
            c1 = cp_v[2 * ti + 1, pl.ds(lo, 16)]
            tv = t_v[pl.ds(q * 16, 16)]
            pos = jnp.clip(tv, 0, 1) > 0
            dmax = jnp.maximum(c0, c1)
            gat = jnp.where(pos, c1, c0)
            ce = dmax - gat + _log1p_poly(jnp.exp(-jnp.abs(c0 - c1)))
            loss = jnp.where(pos, 0.0, ce)
            ce_v[pl.ds(q * 16, 16)] = ce
            loss_v[pl.ds(q * 16, 16)] = loss
            bits = lax.bitcast_convert_type(loss, jnp.int32)
            plsc.addupdate_scatter(h1, [lax.shift_right_logical(bits, 20)],
                                   ones)
            return (npos + jnp.sum(pos.astype(jnp.float32)),
                    spos + jnp.sum(jnp.where(pos, ce, 0.0)))

        num_pos, sum_pos = lax.fori_loop(0, 1024, p1,
                                         (jnp.float32(0.0), jnp.float32(0.0)))
        num_neg = jnp.minimum(jnp.float32(NPR) * num_pos, jnp.float32(16383.0))

        f1, b1, k1, _e1 = _sc_scan_hist(h1, 128, num_neg)

        def p2(q, _):
            bits = lax.bitcast_convert_type(loss_v[pl.ds(q * 16, 16)],
                                            jnp.int32)
            m = lax.shift_right_logical(bits, 20) == b1
            b2v = jnp.bitwise_and(lax.shift_right_logical(bits, 10), 1023)
            plsc.addupdate_scatter(h2, [b2v], ones, mask=m)
            return 0

        lax.fori_loop(0, 1024, p2, 0)
        f2, b2, k2, _e2 = _sc_scan_hist(h2, 64, k1)

        hi21 = jnp.bitwise_or(b1 << 10, b2)

        def p3(q, _):
            bits = lax.bitcast_convert_type(loss_v[pl.ds(q * 16, 16)],
                                            jnp.int32)
            m = lax.shift_right_logical(bits, 10) == hi21
            plsc.addupdate_scatter(h3, [jnp.bitwise_and(bits, 1023)], ones,
                                   mask=m)
            return 0

        lax.fori_loop(0, 1024, p3, 0)
        f3, b3, k3, _e3 = _sc_scan_hist(h3, 64, k2)

        vbits = jnp.where(f3, (b1 << 20) | (b2 << 10) | b3,
                          jnp.int32(0x7F800000))
        vvec = lax.bitcast_convert_type(jnp.broadcast_to(vbits, (16,)),
                                        jnp.float32)

        def p4(q, acc):
            loss = loss_v[pl.ds(q * 16, 16)]
            ce = ce_v[pl.ds(q * 16, 16)]
            return acc + jnp.sum(jnp.where(loss > vvec, ce, 0.0))

        sum_gt = lax.fori_loop(0, 1024, p4, jnp.float32(0.0))

        vstar = jnp.max(vvec)
        tie = jnp.where(jnp.logical_and(f3 > 0, k3 > 0),
                        vstar * k3, 0.0)
        cls_row = sum_pos + sum_gt + tie

        i16 = lax.broadcasted_iota(jnp.int32, (16,), 0)
        stg[pl.ds(0, 16)] = jnp.where(i16 == 0, cls_row,
                                      jnp.where(i16 == 1, num_pos, 0.0))

        def zs(i, _):
            stg[pl.ds((i + 1) * 16, 16)] = jnp.zeros((16,), jnp.float32)
            return 0

        lax.fori_loop(0, 7, zs, 0)
        pltpu.sync_copy(stg, out_hbm.at[row])


def _sc_cls(cp, tg):
    mesh = plsc.VectorSubcoreMesh(core_axis_name="c", subcore_axis_name="s")
    kern = functools.partial(
        pl.kernel,
        mesh=mesh,
        compiler_params=pltpu.CompilerParams(needs_layout_passes=False),
        out_type=jax.ShapeDtypeStruct((8, 128), jnp.float32),
        scratch_types=[
            pltpu.VMEM((256, 128), jnp.float32),
            pltpu.VMEM((16384,), jnp.int32),
            pltpu.VMEM((16384,), jnp.float32),
            pltpu.VMEM((16384,), jnp.float32),
            pltpu.VMEM((2048,), jnp.float32),
            pltpu.VMEM((1056,), jnp.float32),
            pltpu.VMEM((1056,), jnp.float32),
            pltpu.VMEM((128,), jnp.float32),
        ],
    )(_sc_body)
    return kern(cp, tg)


def _tc_body(x_ref, m_ref, lp_ref, lt_ref, t8_ref, out_ref, acc_ref, *,
             grid_i, grid_j, gts_den):
    i = pl.program_id(0)
    j = pl.program_id(1)
    step = i * grid_j + j
    last = grid_i * grid_j - 1

    @pl.when(step == 0)
    def _init():
        acc_ref[0] = 0.0
        acc_ref[1] = 0.0

    x = x_ref[...].reshape(2048, 128)
    bce_sp = jnp.sum(jnp.maximum(x, 0.0) + jnp.log1p(jnp.exp(-jnp.abs(x))))
    xp = x.reshape(1024, 256)
    mf = m_ref[...].reshape(1024, 128) > 0
    gathered = jnp.sum(jnp.where(mf, xp[:, 128:], xp[:, :128]))
    acc_ref[0] = acc_ref[0] + (bce_sp - gathered)

    d = lp_ref[...] - lt_ref[...]
    ad = jnp.abs(d)
    sl1 = jnp.where(ad < 1.0, 0.5 * d * d, ad - 0.5)
    posl = (jnp.clip(t8_ref[...], 0, 1) > 0)[:, None, :]
    acc_ref[1] = acc_ref[1] + jnp.sum(jnp.where(posl, sl1, 0.0))

    @pl.when(step == last)
    def _fin():
        lane = lax.broadcasted_iota(jnp.int32, (1, 128), 1)
        vec = jnp.where(lane == 0, acc_ref[1],
                        jnp.where(lane == 1, acc_ref[0] / gts_den, 0.0))
        out_ref[...] = vec


def _tc_dense(xg, gts_masks, lp, lt, cls_targets, B, K, A, H, W, L):
    GI, GJ = 4, 4
    BB = B // GI
    awb = A // (GI * GJ)
    rows = H * (W // 128) * NC
    in_specs = [
        pl.BlockSpec((1, BB, rows, 128), lambda i, j: (j, i, 0, 0)),
        pl.BlockSpec((BB, H, W), lambda i, j: (i, 0, 0)),
        pl.BlockSpec((B, K, awb), lambda i, j: (0, 0, i * GJ + j)),
        pl.BlockSpec((B, K, awb), lambda i, j: (0, 0, i * GJ + j)),
        pl.BlockSpec((B, awb), lambda i, j: (0, i * GJ + j)),
    ]
    body = functools.partial(_tc_body, grid_i=GI, grid_j=GJ,
                             gts_den=float(L * B * H * W * NC))
    return pl.pallas_call(
        body,
        grid=(GI, GJ),
        in_specs=in_specs,
        out_specs=pl.BlockSpec((1, 128), lambda i, j: (0, 0)),
        out_shape=jax.ShapeDtypeStruct((1, 128), jnp.float32),
        scratch_shapes=[pltpu.SMEM((2,), jnp.float32)],
        compiler_params=pltpu.CompilerParams(
            dimension_semantics=("arbitrary", "arbitrary")),
    )(xg, gts_masks, lp, lt, cls_targets)


def _combine_body(sc_ref, tc_ref, out_ref):
    scv = sc_ref[...]
    l16 = lax.broadcasted_iota(jnp.int32, (8, 128), 1)
    cls_sum = jnp.sum(jnp.where(l16 == 0, scv, 0.0))
    n_tot = jnp.sum(jnp.where(l16 == 1, scv, 0.0))
    tcv = tc_ref[...]
    lane = lax.broadcasted_iota(jnp.int32, (1, 128), 1)
    loc_sum = jnp.sum(jnp.where(lane == 0, tcv, 0.0))
    gts_loss = jnp.sum(jnp.where(lane == 1, tcv, 0.0))
    vec = jnp.where(lane == 0, loc_sum / n_tot,
                    jnp.where(lane == 1, cls_sum / n_tot,
                              jnp.where(lane == 2, gts_loss, 0.0)))
    out_ref[...] = vec


def kernel(loc_preds, loc_targets, cls_preds, cls_targets, global_text_segs,
           gts_masks):
    B, A, K = loc_preds.shape
    L = global_text_segs.shape[0]
    H, W = gts_masks.shape[1:]

    lp = jnp.transpose(loc_preds, (0, 2, 1))
    lt = jnp.transpose(loc_targets, (0, 2, 1))
    cp = cls_preds.reshape(B, A // 128, 128, NC).transpose(0, 1, 3, 2)
    cp = cp.reshape(B, (A // 128) * NC, 128)
    xg = global_text_segs.reshape(L, B, H, W // 128, 128, NC)
    xg = xg.transpose(0, 1, 2, 3, 5, 4).reshape(L, B, H * (W // 128) * NC, 128)

    sc_out = _sc_cls(cp, cls_targets)
    tc_out = _tc_dense(xg, gts_masks, lp, lt, cls_targets, B, K, A, H, W, L)

    out = pl.pallas_call(
        _combine_body,
        out_shape=jax.ShapeDtypeStruct((1, 128), jnp.float32),
    )(sc_out, tc_out)
    return (out[0, 0], out[0, 1], out[0, 2])


# speedup vs baseline: 1.4415x; 1.0209x over previous
import functools

import jax
import jax.numpy as jnp
from jax import lax
from jax.experimental import pallas as pl
from jax.experimental.pallas import tpu as pltpu
from jax.experimental.pallas import tpu_sc as plsc

NC = 2
NPR = 3

_LOG1P = (-0.002260995288484514, 0.015055349381361889, -0.04705113446017369,
          0.09475556299860108, -0.1453396417991718, 0.19351750061793127,
          -0.24872052840014142, 0.33318192091083904, -0.49999062475207734,
          0.9999997699016362, 9.473307568734822e-10)


def _log1p_poly(u):
    acc = jnp.full(u.shape, _LOG1P[0], jnp.float32)
    for c in _LOG1P[1:]:
        acc = acc * u + jnp.float32(c)
    return acc


def _sc_scan_hist(h_ref, nblk, k):
    i16 = lax.broadcasted_iota(jnp.int32, (16,), 0)

    def step(jj, carry):
        before, found, bsel, krem = carry
        j = nblk - 1 - jj
        v = h_ref[pl.ds(j * 16, 16)]
        rv = lax.rev(v, (0,))
        cums = plsc.cumsum(rv)
        tot = jnp.sum(v)
        after = before + tot
        crossed = jnp.logical_and(jnp.logical_and(before < k, after >= k),
                                  found == 0)
        cc = before + cums
        lane = jnp.min(jnp.where(cc >= k, i16, 16))
        sat = jnp.sum(jnp.where(i16 == lane, cc, 0.0))
        hv = jnp.sum(jnp.where(i16 == lane, rv, 0.0))
        nb = j * 16 + (15 - lane)
        bsel = jnp.where(crossed, nb, bsel)
        krem = jnp.where(crossed, k - (sat - hv), krem)
        found = jnp.where(crossed, 1, found)
        return (after, found, bsel, krem)

    endcum, found, bsel, krem = lax.fori_loop(
        0, nblk, step, (jnp.float32(0.0), jnp.int32(0), jnp.int32(0),
                        jnp.float32(0.0)))
    return found, bsel, krem, endcum


def _sc_body(cp_hbm, t_hbm, out_hbm, cp_v, t_v, loss_v, ce_v, h1, h2, h3,
             stg):
    c = lax.axis_index("c")
    s = lax.axis_index("s")

    @pl.when(jnp.logical_and(c == 0, s < 8))
    def _run():
        row = s
        pltpu.sync_copy(cp_hbm.at[row], cp_v)
        pltpu.sync_copy(t_hbm.at[row], t_v)

        zf = jnp.zeros((16,), jnp.float32)

        def zloop(i, _):
            h1[pl.ds(i * 16, 16)] = zf

            @pl.when(i < 66)
            def _z2():
                h2[pl.ds(i * 16, 16)] = zf
                h3[pl.ds(i * 16, 16)] = zf

            return 0

        lax.fori_loop(0, 128, zloop, 0)
        ones = jnp.ones((16,), jnp.float32)

        def p1(qq, carry):
            npos, spos = carry
            for u in range(4):
                q = qq * 4 + u
                ti = q >> 3
                lo = (q & 7) * 16
                c0 = cp_v[2 * ti, pl.ds(lo, 16)]
                c1 = cp_v[2 * ti + 1, pl.ds(lo, 16)]
                tv = t_v[pl.ds(q * 16, 16)]
                pos = jnp.clip(tv, 0, 1) > 0
                dmax = jnp.maximum(c0, c1)
                gat = jnp.where(pos, c1, c0)
                ce = dmax - gat + _log1p_poly(jnp.exp(-jnp.abs(c0 - c1)))
                loss = jnp.where(pos, 0.0, ce)
                ce_v[pl.ds(q * 16, 16)] = ce
                loss_v[pl.ds(q * 16, 16)] = loss
                bits = lax.bitcast_convert_type(loss, jnp.int32)
                plsc.addupdate_scatter(
                    h1, [lax.shift_right_logical(bits, 20)], ones)
                npos = npos + jnp.sum(pos.astype(jnp.float32))
                spos = spos + jnp.sum(jnp.where(pos, ce, 0.0))
            return (npos, spos)

        num_pos, sum_pos = lax.fori_loop(0, 256, p1,
                                         (jnp.float32(0.0), jnp.float32(0.0)))
        num_neg = jnp.minimum(jnp.float32(NPR) * num_pos, jnp.float32(16383.0))

        f1, b1, k1, _e1 = _sc_scan_hist(h1, 128, num_neg)

        def p2(q, _):
            bits = lax.bitcast_convert_type(loss_v[pl.ds(q * 16, 16)],
                                            jnp.int32)
            m = lax.shift_right_logical(bits, 20) == b1
            b2v = jnp.bitwise_and(lax.shift_right_logical(bits, 10), 1023)
            plsc.addupdate_scatter(h2, [b2v], ones, mask=m)
            return 0

        lax.fori_loop(0, 1024, p2, 0)
        f2, b2, k2, _e2 = _sc_scan_hist(h2, 64, k1)

        hi21 = jnp.bitwise_or(b1 << 10, b2)

        def p3(q, _):
            bits = lax.bitcast_convert_type(loss_v[pl.ds(q * 16, 16)],
                                            jnp.int32)
            m = lax.shift_right_logical(bits, 10) == hi21
            plsc.addupdate_scatter(h3, [jnp.bitwise_and(bits, 1023)], ones,
                                   mask=m)
            return 0

        lax.fori_loop(0, 1024, p3, 0)
        f3, b3, k3, _e3 = _sc_scan_hist(h3, 64, k2)

        vbits = jnp.where(f3, (b1 << 20) | (b2 << 10) | b3,
                          jnp.int32(0x7F800000))
        vvec = lax.bitcast_convert_type(jnp.broadcast_to(vbits, (16,)),
                                        jnp.float32)

        def p4(qq, acc):
            for u in range(4):
                q = qq * 4 + u
                loss = loss_v[pl.ds(q * 16, 16)]
                ce = ce_v[pl.ds(q * 16, 16)]
                acc = acc + jnp.sum(jnp.where(loss > vvec, ce, 0.0))
            return acc

        sum_gt = lax.fori_loop(0, 256, p4, jnp.float32(0.0))

        vstar = jnp.max(vvec)
        tie = jnp.where(jnp.logical_and(f3 > 0, k3 > 0),
                        vstar * k3, 0.0)
        cls_row = sum_pos + sum_gt + tie

        i16 = lax.broadcasted_iota(jnp.int32, (16,), 0)
        stg[pl.ds(0, 16)] = jnp.where(i16 == 0, cls_row,
                                      jnp.where(i16 == 1, num_pos, 0.0))

        def zs(i, _):
            stg[pl.ds((i + 1) * 16, 16)] = jnp.zeros((16,), jnp.float32)
            return 0

        lax.fori_loop(0, 7, zs, 0)
        pltpu.sync_copy(stg, out_hbm.at[row])


def _sc_cls(cp, tg):
    mesh = plsc.VectorSubcoreMesh(core_axis_name="c", subcore_axis_name="s")
    kern = functools.partial(
        pl.kernel,
        mesh=mesh,
        compiler_params=pltpu.CompilerParams(needs_layout_passes=False),
        out_type=jax.ShapeDtypeStruct((8, 128), jnp.float32),
        scratch_types=[
            pltpu.VMEM((256, 128), jnp.float32),
            pltpu.VMEM((16384,), jnp.int32),
            pltpu.VMEM((16384,), jnp.float32),
            pltpu.VMEM((16384,), jnp.float32),
            pltpu.VMEM((2048,), jnp.float32),
            pltpu.VMEM((1056,), jnp.float32),
            pltpu.VMEM((1056,), jnp.float32),
            pltpu.VMEM((128,), jnp.float32),
        ],
    )(_sc_body)
    return kern(cp, tg)


def _tc_body(x_ref, m_ref, lp_ref, lt_ref, t8_ref, out_ref, acc_ref, *,
             grid_i, grid_j, gts_den):
    i = pl.program_id(0)
    j = pl.program_id(1)
    step = i * grid_j + j
    last = grid_i * grid_j - 1

    @pl.when(step == 0)
    def _init():
        acc_ref[0] = 0.0
        acc_ref[1] = 0.0

    x = x_ref[...].reshape(2048, 128)
    bce_sp = jnp.sum(jnp.maximum(x, 0.0) + jnp.log1p(jnp.exp(-jnp.abs(x))))
    xp = x.reshape(1024, 256)
    mf = m_ref[...].reshape(1024, 128) > 0
    gathered = jnp.sum(jnp.where(mf, xp[:, 128:], xp[:, :128]))
    acc_ref[0] = acc_ref[0] + (bce_sp - gathered)

    d = lp_ref[...] - lt_ref[...]
    ad = jnp.abs(d)
    sl1 = jnp.where(ad < 1.0, 0.5 * d * d, ad - 0.5)
    posl = (jnp.clip(t8_ref[...], 0, 1) > 0)[:, None, :]
    acc_ref[1] = acc_ref[1] + jnp.sum(jnp.where(posl, sl1, 0.0))

    @pl.when(step == last)
    def _fin():
        lane = lax.broadcasted_iota(jnp.int32, (1, 128), 1)
        vec = jnp.where(lane == 0, acc_ref[1],
                        jnp.where(lane == 1, acc_ref[0] / gts_den, 0.0))
        out_ref[...] = vec


def _tc_dense(xg, gts_masks, lp, lt, cls_targets, B, K, A, H, W, L):
    GI, GJ = 4, 4
    BB = B // GI
    awb = A // (GI * GJ)
    rows = H * (W // 128) * NC
    in_specs = [
        pl.BlockSpec((1, BB, rows, 128), lambda i, j: (j, i, 0, 0)),
        pl.BlockSpec((BB, H, W), lambda i, j: (i, 0, 0)),
        pl.BlockSpec((B, K, awb), lambda i, j: (0, 0, i * GJ + j)),
        pl.BlockSpec((B, K, awb), lambda i, j: (0, 0, i * GJ + j)),
        pl.BlockSpec((B, awb), lambda i, j: (0, i * GJ + j)),
    ]
    body = functools.partial(_tc_body, grid_i=GI, grid_j=GJ,
                             gts_den=float(L * B * H * W * NC))
    return pl.pallas_call(
        body,
        grid=(GI, GJ),
        in_specs=in_specs,
        out_specs=pl.BlockSpec((1, 128), lambda i, j: (0, 0)),
        out_shape=jax.ShapeDtypeStruct((1, 128), jnp.float32),
        scratch_shapes=[pltpu.SMEM((2,), jnp.float32)],
        compiler_params=pltpu.CompilerParams(
            dimension_semantics=("arbitrary", "arbitrary")),
    )(xg, gts_masks, lp, lt, cls_targets)


def _combine_body(sc_ref, tc_ref, out_ref):
    scv = sc_ref[...]
    l16 = lax.broadcasted_iota(jnp.int32, (8, 128), 1)
    cls_sum = jnp.sum(jnp.where(l16 == 0, scv, 0.0))
    n_tot = jnp.sum(jnp.where(l16 == 1, scv, 0.0))
    tcv = tc_ref[...]
    lane = lax.broadcasted_iota(jnp.int32, (1, 128), 1)
    loc_sum = jnp.sum(jnp.where(lane == 0, tcv, 0.0))
    gts_loss = jnp.sum(jnp.where(lane == 1, tcv, 0.0))
    vec = jnp.where(lane == 0, loc_sum / n_tot,
                    jnp.where(lane == 1, cls_sum / n_tot,
                              jnp.where(lane == 2, gts_loss, 0.0)))
    out_ref[...] = vec


def kernel(loc_preds, loc_targets, cls_preds, cls_targets, global_text_segs,
           gts_masks):
    B, A, K = loc_preds.shape
    L = global_text_segs.shape[0]
    H, W = gts_masks.shape[1:]

    lp = jnp.transpose(loc_preds, (0, 2, 1))
    lt = jnp.transpose(loc_targets, (0, 2, 1))
    cp = cls_preds.reshape(B, A // 128, 128, NC).transpose(0, 1, 3, 2)
    cp = cp.reshape(B, (A // 128) * NC, 128)
    xg = global_text_segs.reshape(L, B, H, W // 128, 128, NC)
    xg = xg.transpose(0, 1, 2, 3, 5, 4).reshape(L, B, H * (W // 128) * NC, 128)

    sc_out = _sc_cls(cp, cls_targets)
    tc_out = _tc_dense(xg, gts_masks, lp, lt, cls_targets, B, K, A, H, W, L)

    out = pl.pallas_call(
        _combine_body,
        out_shape=jax.ShapeDtypeStruct((1, 128), jnp.float32),
    )(sc_out, tc_out)
    return (out[0, 0], out[0, 1], out[0, 2])
